# baseline (speedup 1.0000x reference)
import functools

import numpy as np
import jax
import jax.numpy as jnp
from jax import lax
from jax.experimental import pallas as pl
from jax.experimental.pallas import tpu as pltpu
from jax.experimental.pallas import tpu_sc as plsc

N = 10000
E = 640000
C = 108
G = 128
PROJ = 256

NTILES = 16
RPT = 632
P = NTILES * RPT
DUMMY = 10008
CH = 128
RING = 32

_CONST = None


def _tf2x32(k1, k2, c1, c2):
    rot0 = (13, 15, 26, 6)
    rot1 = (17, 29, 16, 24)
    ks = (k1, k2, np.uint32(k1 ^ k2 ^ np.uint32(0x1BD11BDA)))
    x0 = (c1 + ks[0]).astype(np.uint32)
    x1 = (c2 + ks[1]).astype(np.uint32)

    def rounds(x0, x1, rots):
        for r in rots:
            x0 = (x0 + x1).astype(np.uint32)
            x1 = ((x1 << np.uint32(r)) | (x1 >> np.uint32(32 - r))).astype(np.uint32)
            x1 = x0 ^ x1
        return x0, x1

    for i, (rots, ka, kb) in enumerate((
            (rot0, ks[1], ks[2]), (rot1, ks[2], ks[0]), (rot0, ks[0], ks[1]),
            (rot1, ks[1], ks[2]), (rot0, ks[2], ks[0]))):
        x0, x1 = rounds(x0, x1, rots)
        x0 = (x0 + ka).astype(np.uint32)
        x1 = (x1 + kb + np.uint32(i + 1)).astype(np.uint32)
    return x0, x1


def _np_uniform01(key, n):
    b1, b2 = _tf2x32(key[0], key[1], np.zeros(n, np.uint32),
                     np.arange(n, dtype=np.uint32))
    bits = b1 ^ b2
    f = ((bits >> np.uint32(9)) | np.uint32(0x3F800000)).view(np.float32)
    return f - np.float32(1.0)


def _consts():
    global _CONST
    if _CONST is not None:
        return _CONST
    b1, b2 = _tf2x32(np.uint32(0), np.uint32(42), np.zeros(4, np.uint32),
                     np.arange(4, dtype=np.uint32))
    subkeys = [(b1[i], b2[i]) for i in range(4)]
    keep1 = _np_uniform01(subkeys[0], E) < 0.5
    keep2 = _np_uniform01(subkeys[1], E) < 0.5
    mask1 = _np_uniform01(subkeys[2], C) < 0.5
    idx1 = np.where(keep1)[0].astype(np.int64)
    idx2 = np.where(keep2)[0].astype(np.int64)
    kc = np.where(~mask1)[0].astype(np.int32)

    k = max(-(-len(i) // (NTILES * CH)) for i in (idx1, idx2))
    k += k & 1
    oc = k * CH
    rows_e = E // 16 + 1
    spans = []
    pps = []
    for pos in (idx1, idx2):
        pp = np.full((NTILES, oc), E, np.int64)
        pp.ravel()[:len(pos)] = pos
        r0 = pp[:, 0] // 16
        spans.append((pp[:, -1] // 16 - r0 + 1).max())
        pps.append(pp)
    rmax = _ceil_to(int(max(spans)), CH)
    rq = rmax // CH
    lidx = np.empty((2, NTILES, k, CH), np.int32)
    ridx = np.empty((2, NTILES, rq, CH), np.int32)
    for b, pp in enumerate(pps):
        r0p = np.minimum(pp[:, 0] // 16, rows_e - rmax)
        lidx[b] = (pp - (r0p * 16)[:, None]).reshape(NTILES, k, CH)
        ridx[b] = np.minimum(r0p[:, None] + np.arange(rmax)[None, :],
                             rows_e - 1).reshape(NTILES, rq, CH)
    _CONST = (kc, k, rq, lidx, ridx)
    return _CONST


def _ceil_to(v, m):
    return -(-v // m) * m


def _pad_edges(v, k):
    L = NTILES * k * CH
    v = jnp.concatenate([v, jnp.full((L - v.shape[0],), DUMMY, jnp.int32)])
    return v.reshape(NTILES, k, CH)


_MESH = dict(core_axis_name="c", subcore_axis_name="s")
_SC_PARAMS = pltpu.CompilerParams(use_tc_tiling_on_sc=False)
_SC_PARAMS_NOLAYOUT = pltpu.CompilerParams(use_tc_tiling_on_sc=False,
                                           needs_layout_passes=False)


def _sc_compact(edges2, lidx2, ridx2, k, rq):
    rb = rq * CH

    @functools.partial(
        pl.kernel,
        out_type=[jax.ShapeDtypeStruct((2, NTILES, k, CH), jnp.int32),
                  jax.ShapeDtypeStruct((2, NTILES, k, CH), jnp.int32)],
        mesh=plsc.VectorSubcoreMesh(**_MESH),
        compiler_params=_SC_PARAMS_NOLAYOUT,
        scratch_types=[
            pltpu.VMEM((rq, CH), jnp.int32),
            pltpu.VMEM((k, CH), jnp.int32),
            pltpu.VMEM((rb, 16), jnp.int32),
            pltpu.VMEM((k, CH), jnp.int32),
            pltpu.SemaphoreType.DMA,
        ],
    )
    def comp(e_hbm, li_hbm, ri_hbm, src_out, dst_out, vri, vli, reg, vout, sem):
        c = lax.axis_index("c")
        s = lax.axis_index("s")
        pltpu.sync_copy(ri_hbm.at[c].at[s], vri)
        pltpu.sync_copy(li_hbm.at[c].at[s], vli)
        for p, out in ((0, src_out), (1, dst_out)):
            @pl.loop(0, rq)
            def _(q):
                pltpu.async_copy(e_hbm.at[p].at[vri.at[q]],
                                 reg.at[pl.ds(q * CH, CH)], sem).wait()

            @pl.loop(0, k)
            def _(j):
                @pl.loop(0, CH // 16)
                def _(q2):
                    v = vli.at[j][pl.ds(q2 * 16, 16)]
                    row = lax.shift_right_logical(v, 4)
                    lane = lax.bitwise_and(v, 15)
                    vout.at[j][pl.ds(q2 * 16, 16)] = plsc.load_gather(
                        reg, [row, lane])

            pltpu.sync_copy(vout, out.at[c].at[s])

    return comp(edges2, lidx2, ridx2)


def _sc_hist(dst2, ones, zeros, k):
    @functools.partial(
        pl.kernel,
        out_type=jax.ShapeDtypeStruct((2 * P, 16), jnp.float32),
        mesh=plsc.VectorSubcoreMesh(**_MESH),
        compiler_params=_SC_PARAMS,
        scratch_types=[
            pltpu.VMEM((k, CH), jnp.int32),
            pltpu.VMEM((CH, 16), jnp.float32),
            pltpu.VMEM((RPT, 16), jnp.float32),
            pltpu.VMEM_SHARED((P, 16), jnp.float32),
        ],
    )
    def hist(dst_hbm, ones_hbm, zeros_hbm, out_hbm, idst, vones, vzeros, acc):
        c = lax.axis_index("c")
        s = lax.axis_index("s")
        pltpu.sync_copy(dst_hbm.at[c].at[s], idst)
        pltpu.sync_copy(ones_hbm, vones)
        pltpu.sync_copy(zeros_hbm, vzeros)
        pltpu.sync_copy(vzeros, acc.at[pl.ds(s * RPT, RPT)])
        plsc.subcore_barrier()

        @pl.loop(0, k)
        def _(j):
            pltpu.sync_copy(vones, acc.at[idst.at[j]], add=True)

        plsc.subcore_barrier()
        pltpu.sync_copy(acc.at[pl.ds(s * RPT, RPT)],
                        out_hbm.at[pl.ds(c * P + s * RPT, RPT)])

    return hist(dst2, ones, zeros)


def _sc_prop(g2, src2, dstq, dh, k):
    @functools.partial(
        pl.kernel,
        out_type=jax.ShapeDtypeStruct((2 * P, dh), jnp.float32),
        mesh=plsc.VectorSubcoreMesh(**_MESH),
        compiler_params=_SC_PARAMS,
        scratch_types=[
            pltpu.VMEM((k, CH), jnp.int32),
            pltpu.VMEM((k, CH), jnp.int32),
            pltpu.VMEM((CH, dh), jnp.float32),
            pltpu.VMEM_SHARED((P, dh), jnp.float32),
            pltpu.SemaphoreType.DMA,
        ],
    )
    def prop(g_hbm, src_hbm, dst_hbm, out_hbm, isrc, idst, buf, acc, sem):
        c = lax.axis_index("c")
        s = lax.axis_index("s")
        pltpu.sync_copy(src_hbm.at[c].at[s], isrc)
        pltpu.sync_copy(dst_hbm.at[s], idst)
        pltpu.sync_copy(g_hbm.at[pl.ds(c * P + s * RPT, RPT)],
                        acc.at[pl.ds(s * RPT, RPT)])
        plsc.subcore_barrier()

        @pl.loop(0, k)
        def _(j):
            pltpu.async_copy(g_hbm.at[isrc.at[j]], buf, sem).wait()
            pltpu.sync_copy(buf, acc.at[idst.at[j]], add=True)

        plsc.subcore_barrier()
        pltpu.sync_copy(acc.at[pl.ds(s * RPT, RPT)],
                        out_hbm.at[pl.ds(c * P + s * RPT, RPT)])

    return prop(g2, src2, dstq)


def _tc_prep(xa2, xb2, cnt_a, cnt_b, da_out_w, db_out_w):
    B = RPT
    wa = xa2.shape[2]
    wb = xb2.shape[2]

    def body(xa_ref, xb_ref, ca_ref, cb_ref, ga_ref, gb_ref, da_ref, db_ref):
        da = lax.rsqrt(1.0 + ca_ref[:, 0:1])
        db = lax.rsqrt(1.0 + cb_ref[:, 0:1])
        ga_ref[...] = (da * xa_ref[0])[None]
        gb_ref[...] = (db * xb_ref[0])[None]
        da_ref[...] = da
        db_ref[...] = db

    return pl.pallas_call(
        body,
        grid=(2, NTILES),
        in_specs=[
            pl.BlockSpec((1, B, wa), lambda c, i: (c, i, 0)),
            pl.BlockSpec((1, B, wb), lambda c, i: (c, i, 0)),
            pl.BlockSpec((B, 16), lambda c, i: (i, 0)),
            pl.BlockSpec((B, 16), lambda c, i: (i, 0)),
        ],
        out_specs=[
            pl.BlockSpec((1, B, wa), lambda c, i: (c, i, 0)),
            pl.BlockSpec((1, B, wb), lambda c, i: (c, i, 0)),
            pl.BlockSpec((B, 1), lambda c, i: (i, 0)),
            pl.BlockSpec((B, 1), lambda c, i: (i, 0)),
        ],
        out_shape=[
            jax.ShapeDtypeStruct((2, P, wa), jnp.float32),
            jax.ShapeDtypeStruct((2, P, wb), jnp.float32),
            jax.ShapeDtypeStruct((P, 1), jnp.float32),
            jax.ShapeDtypeStruct((P, 1), jnp.float32),
        ],
    )(xa2, xb2, cnt_a, cnt_b)


def _tc_mid(s2, dinv, w2, b2, a, dout):
    B = RPT
    dinh = s2.shape[2]
    din = 2 * dinh
    douth = dout // 2

    def body(s_ref, d_ref, w_ref, b_ref, a_ref, o_ref):
        d = d_ref[...]
        t = jnp.concatenate([s_ref[0], s_ref[1]], axis=1) * d
        y = jnp.dot(t, w_ref[0], preferred_element_type=jnp.float32) + b_ref[0]
        h = jnp.where(y >= 0, y, a_ref[0, 0] * y)
        o_ref[...] = (d * h)[None]

    return pl.pallas_call(
        body,
        grid=(2, NTILES),
        in_specs=[
            pl.BlockSpec((2, B, dinh), lambda c, i: (0, i, 0)),
            pl.BlockSpec((B, 1), lambda c, i: (i, 0)),
            pl.BlockSpec((1, din, douth), lambda c, i: (c, 0, 0)),
            pl.BlockSpec((1, 1, douth), lambda c, i: (c, 0, 0)),
            pl.BlockSpec((1, 1), lambda c, i: (0, 0)),
        ],
        out_specs=pl.BlockSpec((1, B, douth), lambda c, i: (c, i, 0)),
        out_shape=jax.ShapeDtypeStruct((2, P, douth), jnp.float32),
    )(s2, dinv, w2, b2, a)


def _tc_final(s2, dinv, batchp, w3, b3, a, wp, bp):
    B = RPT
    dinh = s2.shape[2]
    F = w3.shape[1]

    def body(s_ref, d_ref, bt_ref, w_ref, b_ref, a_ref, wp_ref, bp_ref,
             z_ref, acc_ref):
        i = pl.program_id(0)

        @pl.when(i == 0)
        def _():
            acc_ref[...] = jnp.full((G, F), -jnp.inf, jnp.float32)

        t = jnp.concatenate([s_ref[0], s_ref[1]], axis=1) * d_ref[...]
        y = jnp.dot(t, w_ref[...], preferred_element_type=jnp.float32) + b_ref[...]
        h = jnp.where(y >= 0, y, a_ref[0, 0] * y)
        bt = bt_ref[...]
        g_lo = bt[0, 0]
        g_hi = bt[B - 1, 0]
        rows = lax.broadcasted_iota(jnp.int32, (G, 1), 0)

        def gbody(g, carry):
            m = bt == g
            contrib = jnp.max(jnp.where(m, h, -jnp.inf), axis=0, keepdims=True)
            upd = jnp.maximum(acc_ref[...], contrib)
            acc_ref[...] = jnp.where(rows == g, upd, acc_ref[...])
            return carry

        lax.fori_loop(g_lo, g_hi + 1, gbody, 0)

        @pl.when(i == NTILES - 1)
        def _():
            z_ref[...] = jnp.dot(acc_ref[...], wp_ref[...],
                                 preferred_element_type=jnp.float32) + bp_ref[...]

    return pl.pallas_call(
        body,
        grid=(NTILES,),
        in_specs=[
            pl.BlockSpec((2, B, dinh), lambda i: (0, i, 0)),
            pl.BlockSpec((B, 1), lambda i: (i, 0)),
            pl.BlockSpec((B, 1), lambda i: (i, 0)),
            pl.BlockSpec((2 * dinh, F), lambda i: (0, 0)),
            pl.BlockSpec((1, F), lambda i: (0, 0)),
            pl.BlockSpec((1, 1), lambda i: (0, 0)),
            pl.BlockSpec((F, PROJ), lambda i: (0, 0)),
            pl.BlockSpec((1, PROJ), lambda i: (0, 0)),
        ],
        out_specs=pl.BlockSpec((G, PROJ), lambda i: (0, 0)),
        out_shape=jax.ShapeDtypeStruct((G, PROJ), jnp.float32),
        scratch_shapes=[pltpu.VMEM((G, F), jnp.float32)],
    )(s2, dinv, batchp, w3, b3, a, wp, bp)


def _split_cols(m, dpad):
    m = jnp.pad(m, ((0, 0), (0, dpad - m.shape[1])))
    return m.reshape(P, 2, dpad // 2).transpose(1, 0, 2)


def _split_w(w, b, din_pad, dout_pad):
    w = jnp.pad(w, ((0, din_pad - w.shape[0]), (0, dout_pad - w.shape[1])))
    b = jnp.pad(b, (0, dout_pad - b.shape[0]))
    douth = dout_pad // 2
    w2 = w.reshape(din_pad, 2, douth).transpose(1, 0, 2)
    b2 = b.reshape(1, 2, douth).transpose(1, 0, 2)
    return w2, b2


def kernel(x, edge_index, batch, W1a, b1a, W2a, b2a, W3a, b3a, a1,
           W1b, b1b, W2b, b2b, W3b, b3b, a2, Wp, bp):
    kc, k, rq, lidx, ridx = _consts()
    ka = kb = kh = k

    d1a = max(32, _ceil_to(len(kc), 32))
    d1b = 128
    d2 = 128
    d3 = 224

    pad16 = jnp.full((16,), DUMMY, jnp.int32)
    edges2 = jnp.stack([
        jnp.concatenate([edge_index[0].astype(jnp.int32), pad16]),
        jnp.concatenate([edge_index[1].astype(jnp.int32), pad16]),
    ]).reshape(2, E // 16 + 1, 16)
    srcq, dstq = _sc_compact(edges2, jnp.asarray(lidx), jnp.asarray(ridx),
                             k, rq)
    src_a = jnp.stack([srcq[0], srcq[0] + P])
    src_b = jnp.stack([srcq[1], srcq[1] + P])
    dst_a, dst_b = dstq[0], dstq[1]

    ones = jnp.ones((CH, 16), jnp.float32)
    zeros = jnp.zeros((RPT, 16), jnp.float32)

    cnt = _sc_hist(dstq, ones, zeros, kh)
    cnt_a, cnt_b = cnt[:P], cnt[P:]

    xp = jnp.pad(x, ((0, P - N), (0, 0)))
    xa2 = _split_cols(jnp.take(xp, kc, axis=1), d1a)
    xb2 = _split_cols(xp, d1b)

    g0a, g0b, dinv_a, dinv_b = _tc_prep(xa2, xb2, cnt_a, cnt_b, None, None)

    batchp = jnp.pad(batch.astype(jnp.int32), (0, P - N),
                     constant_values=G).reshape(P, 1)

    w1a2, b1a2 = _split_w(jnp.take(W1a, kc, axis=0), b1a, d1a, d2)
    w1b2, b1b2 = _split_w(W1b, b1b, d1b, d2)
    w2a2, b2a2 = _split_w(W2a, b2a, d2, d3)
    w2b2, b2b2 = _split_w(W2b, b2b, d2, d3)
    a1r = a1.reshape(1, 1)
    a2r = a2.reshape(1, 1)
    w3a = jnp.pad(W3a, ((0, d3 - W3a.shape[0]), (0, 0)))
    w3b = jnp.pad(W3b, ((0, d3 - W3b.shape[0]), (0, 0)))
    b3ar = b3a.reshape(1, -1)
    b3br = b3b.reshape(1, -1)
    wpr = Wp
    bpr = bp.reshape(1, -1)

    def branch(g0, src2, dstb, dinv, w1, b1, w2, b2, w3, b3, ar):
        s1 = _sc_prop(g0.reshape(2 * P, -1), src2, dstb, g0.shape[2], k)
        g1 = _tc_mid(s1.reshape(2, P, -1), dinv, w1, b1, ar, d2)
        s2 = _sc_prop(g1.reshape(2 * P, -1), src2, dstb, d2 // 2, k)
        g2 = _tc_mid(s2.reshape(2, P, -1), dinv, w2, b2, ar, d3)
        s3 = _sc_prop(g2.reshape(2 * P, -1), src2, dstb, d3 // 2, k)
        return _tc_final(s3.reshape(2, P, -1), dinv, batchp, w3, b3, ar,
                         wpr, bpr)

    z1 = branch(g0a, src_a, dst_a, dinv_a, w1a2, b1a2, w2a2, b2a2,
                w3a, b3ar, a1r)
    z2 = branch(g0b, src_b, dst_b, dinv_b, w1b2, b1b2, w2b2, b2b2,
                w3b, b3br, a2r)
    return (z1, z2)


# speedup vs baseline: 1.1599x; 1.1599x over previous
import functools

import numpy as np
import jax
import jax.numpy as jnp
from jax import lax
from jax.experimental import pallas as pl
from jax.experimental.pallas import tpu as pltpu
from jax.experimental.pallas import tpu_sc as plsc

N = 10000
E = 640000
C = 108
G = 128
PROJ = 256

NTILES = 16
RPT = 632
P = NTILES * RPT
DUMMY = 10008
CH = 128
RING = 32

_CONST = None


def _tf2x32(k1, k2, c1, c2):
    rot0 = (13, 15, 26, 6)
    rot1 = (17, 29, 16, 24)
    ks = (k1, k2, np.uint32(k1 ^ k2 ^ np.uint32(0x1BD11BDA)))
    x0 = (c1 + ks[0]).astype(np.uint32)
    x1 = (c2 + ks[1]).astype(np.uint32)

    def rounds(x0, x1, rots):
        for r in rots:
            x0 = (x0 + x1).astype(np.uint32)
            x1 = ((x1 << np.uint32(r)) | (x1 >> np.uint32(32 - r))).astype(np.uint32)
            x1 = x0 ^ x1
        return x0, x1

    for i, (rots, ka, kb) in enumerate((
            (rot0, ks[1], ks[2]), (rot1, ks[2], ks[0]), (rot0, ks[0], ks[1]),
            (rot1, ks[1], ks[2]), (rot0, ks[2], ks[0]))):
        x0, x1 = rounds(x0, x1, rots)
        x0 = (x0 + ka).astype(np.uint32)
        x1 = (x1 + kb + np.uint32(i + 1)).astype(np.uint32)
    return x0, x1


def _np_uniform01(key, n):
    b1, b2 = _tf2x32(key[0], key[1], np.zeros(n, np.uint32),
                     np.arange(n, dtype=np.uint32))
    bits = b1 ^ b2
    f = ((bits >> np.uint32(9)) | np.uint32(0x3F800000)).view(np.float32)
    return f - np.float32(1.0)


def _consts():
    global _CONST
    if _CONST is not None:
        return _CONST
    b1, b2 = _tf2x32(np.uint32(0), np.uint32(42), np.zeros(4, np.uint32),
                     np.arange(4, dtype=np.uint32))
    subkeys = [(b1[i], b2[i]) for i in range(4)]
    keep1 = _np_uniform01(subkeys[0], E) < 0.5
    keep2 = _np_uniform01(subkeys[1], E) < 0.5
    mask1 = _np_uniform01(subkeys[2], C) < 0.5
    idx1 = np.where(keep1)[0].astype(np.int64)
    idx2 = np.where(keep2)[0].astype(np.int64)
    kc = np.where(~mask1)[0].astype(np.int32)

    k = max(-(-len(i) // (NTILES * CH)) for i in (idx1, idx2))
    oc = k * CH
    rows_e = E // 16 + 1
    spans = []
    pps = []
    for pos in (idx1, idx2):
        pp = np.full((NTILES, oc), E, np.int64)
        pp.ravel()[:len(pos)] = pos
        r0 = pp[:, 0] // 16
        spans.append((pp[:, -1] // 16 - r0 + 1).max())
        pps.append(pp)
    rmax = _ceil_to(int(max(spans)), CH)
    rq = rmax // CH
    lidx = np.empty((2, NTILES, k, CH), np.int32)
    ridx = np.empty((2, NTILES, rq, CH), np.int32)
    for b, pp in enumerate(pps):
        r0p = np.minimum(pp[:, 0] // 16, rows_e - rmax)
        lidx[b] = (pp - (r0p * 16)[:, None]).reshape(NTILES, k, CH)
        ridx[b] = np.minimum(r0p[:, None] + np.arange(rmax)[None, :],
                             rows_e - 1).reshape(NTILES, rq, CH)
    _CONST = (kc, k, rq, lidx, ridx)
    return _CONST


def _ceil_to(v, m):
    return -(-v // m) * m


def _pad_edges(v, k):
    L = NTILES * k * CH
    v = jnp.concatenate([v, jnp.full((L - v.shape[0],), DUMMY, jnp.int32)])
    return v.reshape(NTILES, k, CH)


_MESH = dict(core_axis_name="c", subcore_axis_name="s")
_SC_PARAMS = pltpu.CompilerParams(use_tc_tiling_on_sc=False)
_SC_PARAMS_NOLAYOUT = pltpu.CompilerParams(use_tc_tiling_on_sc=False,
                                           needs_layout_passes=False)


def _sc_compact(edges2, lidx2, ridx2, k, rq):
    rb = rq * CH

    @functools.partial(
        pl.kernel,
        out_type=[jax.ShapeDtypeStruct((2, NTILES, k, CH), jnp.int32),
                  jax.ShapeDtypeStruct((2, NTILES, k, CH), jnp.int32)],
        mesh=plsc.VectorSubcoreMesh(**_MESH),
        compiler_params=_SC_PARAMS_NOLAYOUT,
        scratch_types=[
            pltpu.VMEM((rq, CH), jnp.int32),
            pltpu.VMEM((k, CH), jnp.int32),
            pltpu.VMEM((rb, 16), jnp.int32),
            pltpu.VMEM((k, CH), jnp.int32),
            pltpu.SemaphoreType.DMA,
        ],
    )
    def comp(e_hbm, li_hbm, ri_hbm, src_out, dst_out, vri, vli, reg, vout, sem):
        c = lax.axis_index("c")
        s = lax.axis_index("s")
        pltpu.sync_copy(ri_hbm.at[c].at[s], vri)
        pltpu.sync_copy(li_hbm.at[c].at[s], vli)
        for p, out in ((0, src_out), (1, dst_out)):
            @pl.loop(0, rq)
            def _(q):
                pltpu.async_copy(e_hbm.at[p].at[vri.at[q]],
                                 reg.at[pl.ds(q * CH, CH)], sem).wait()

            @pl.loop(0, k)
            def _(j):
                @pl.loop(0, CH // 16)
                def _(q2):
                    v = vli.at[j][pl.ds(q2 * 16, 16)]
                    row = lax.shift_right_logical(v, 4)
                    lane = lax.bitwise_and(v, 15)
                    vout.at[j][pl.ds(q2 * 16, 16)] = plsc.load_gather(
                        reg, [row, lane])

            pltpu.sync_copy(vout, out.at[c].at[s])

    return comp(edges2, lidx2, ridx2)


def _sc_hist(dst2, ones, zeros, k):
    @functools.partial(
        pl.kernel,
        out_type=jax.ShapeDtypeStruct((2 * P, 16), jnp.float32),
        mesh=plsc.VectorSubcoreMesh(**_MESH),
        compiler_params=_SC_PARAMS,
        scratch_types=[
            pltpu.VMEM((k, CH), jnp.int32),
            pltpu.VMEM((CH, 16), jnp.float32),
            pltpu.VMEM((RPT, 16), jnp.float32),
            pltpu.VMEM_SHARED((P, 16), jnp.float32),
        ],
    )
    def hist(dst_hbm, ones_hbm, zeros_hbm, out_hbm, idst, vones, vzeros, acc):
        c = lax.axis_index("c")
        s = lax.axis_index("s")
        pltpu.sync_copy(dst_hbm.at[c].at[s], idst)
        pltpu.sync_copy(ones_hbm, vones)
        pltpu.sync_copy(zeros_hbm, vzeros)
        pltpu.sync_copy(vzeros, acc.at[pl.ds(s * RPT, RPT)])
        plsc.subcore_barrier()

        @pl.loop(0, k)
        def _(j):
            pltpu.sync_copy(vones, acc.at[idst.at[j]], add=True)

        plsc.subcore_barrier()
        pltpu.sync_copy(acc.at[pl.ds(s * RPT, RPT)],
                        out_hbm.at[pl.ds(c * P + s * RPT, RPT)])

    return hist(dst2, ones, zeros)


def _sc_prop(g2, src2, dstq, dh, k):
    @functools.partial(
        pl.kernel,
        out_type=jax.ShapeDtypeStruct((2 * P, dh), jnp.float32),
        mesh=plsc.VectorSubcoreMesh(**_MESH),
        compiler_params=_SC_PARAMS,
        scratch_types=[
            pltpu.VMEM((k, CH), jnp.int32),
            pltpu.VMEM((k, CH), jnp.int32),
            pltpu.VMEM((CH, dh), jnp.float32),
            pltpu.VMEM_SHARED((P, dh), jnp.float32),
            pltpu.SemaphoreType.DMA,
        ],
    )
    def prop(g_hbm, src_hbm, dst_hbm, out_hbm, isrc, idst, buf, acc, sem):
        c = lax.axis_index("c")
        s = lax.axis_index("s")
        pltpu.sync_copy(src_hbm.at[c].at[s], isrc)
        pltpu.sync_copy(dst_hbm.at[s], idst)
        pltpu.sync_copy(g_hbm.at[pl.ds(c * P + s * RPT, RPT)],
                        acc.at[pl.ds(s * RPT, RPT)])
        plsc.subcore_barrier()

        @pl.loop(0, k)
        def _(j):
            pltpu.async_copy(g_hbm.at[isrc.at[j]], buf, sem).wait()
            pltpu.sync_copy(buf, acc.at[idst.at[j]], add=True)

        plsc.subcore_barrier()
        pltpu.sync_copy(acc.at[pl.ds(s * RPT, RPT)],
                        out_hbm.at[pl.ds(c * P + s * RPT, RPT)])

    return prop(g2, src2, dstq)


def _tc_prep(xa2, xb2, cnt_a, cnt_b, da_out_w, db_out_w):
    B = RPT
    wa = xa2.shape[2]
    wb = xb2.shape[2]

    def body(xa_ref, xb_ref, ca_ref, cb_ref, ga_ref, gb_ref, da_ref, db_ref):
        da = lax.rsqrt(1.0 + ca_ref[:, 0:1])
        db = lax.rsqrt(1.0 + cb_ref[:, 0:1])
        ga_ref[...] = (da * xa_ref[0])[None]
        gb_ref[...] = (db * xb_ref[0])[None]
        da_ref[...] = da
        db_ref[...] = db

    return pl.pallas_call(
        body,
        grid=(2, NTILES),
        in_specs=[
            pl.BlockSpec((1, B, wa), lambda c, i: (c, i, 0)),
            pl.BlockSpec((1, B, wb), lambda c, i: (c, i, 0)),
            pl.BlockSpec((B, 16), lambda c, i: (i, 0)),
            pl.BlockSpec((B, 16), lambda c, i: (i, 0)),
        ],
        out_specs=[
            pl.BlockSpec((1, B, wa), lambda c, i: (c, i, 0)),
            pl.BlockSpec((1, B, wb), lambda c, i: (c, i, 0)),
            pl.BlockSpec((B, 1), lambda c, i: (i, 0)),
            pl.BlockSpec((B, 1), lambda c, i: (i, 0)),
        ],
        out_shape=[
            jax.ShapeDtypeStruct((2, P, wa), jnp.float32),
            jax.ShapeDtypeStruct((2, P, wb), jnp.float32),
            jax.ShapeDtypeStruct((P, 1), jnp.float32),
            jax.ShapeDtypeStruct((P, 1), jnp.float32),
        ],
    )(xa2, xb2, cnt_a, cnt_b)


def _tc_mid(s2, dinv, w2, b2, a, dout):
    B = RPT
    dinh = s2.shape[2]
    din = 2 * dinh
    douth = dout // 2

    def body(s_ref, d_ref, w_ref, b_ref, a_ref, o_ref):
        d = d_ref[...]
        t = jnp.concatenate([s_ref[0], s_ref[1]], axis=1) * d
        y = jnp.dot(t, w_ref[0], preferred_element_type=jnp.float32) + b_ref[0]
        h = jnp.where(y >= 0, y, a_ref[0, 0] * y)
        o_ref[...] = (d * h)[None]

    return pl.pallas_call(
        body,
        grid=(2, NTILES),
        in_specs=[
            pl.BlockSpec((2, B, dinh), lambda c, i: (0, i, 0)),
            pl.BlockSpec((B, 1), lambda c, i: (i, 0)),
            pl.BlockSpec((1, din, douth), lambda c, i: (c, 0, 0)),
            pl.BlockSpec((1, 1, douth), lambda c, i: (c, 0, 0)),
            pl.BlockSpec((1, 1), lambda c, i: (0, 0)),
        ],
        out_specs=pl.BlockSpec((1, B, douth), lambda c, i: (c, i, 0)),
        out_shape=jax.ShapeDtypeStruct((2, P, douth), jnp.float32),
    )(s2, dinv, w2, b2, a)


def _tc_final(s2, dinv, batchp, w3, b3, a, wp, bp):
    B = RPT
    dinh = s2.shape[2]
    F = w3.shape[1]

    def body(s_ref, d_ref, bt_ref, w_ref, b_ref, a_ref, wp_ref, bp_ref,
             z_ref, acc_ref):
        i = pl.program_id(0)

        @pl.when(i == 0)
        def _():
            acc_ref[...] = jnp.full((G, F), -jnp.inf, jnp.float32)

        t = jnp.concatenate([s_ref[0], s_ref[1]], axis=1) * d_ref[...]
        y = jnp.dot(t, w_ref[...], preferred_element_type=jnp.float32) + b_ref[...]
        h = jnp.where(y >= 0, y, a_ref[0, 0] * y)
        bt = bt_ref[...]
        g_lo = bt[0, 0]
        g_hi = bt[B - 1, 0]
        rows = lax.broadcasted_iota(jnp.int32, (G, 1), 0)

        def gbody(g, carry):
            m = bt == g
            contrib = jnp.max(jnp.where(m, h, -jnp.inf), axis=0, keepdims=True)
            upd = jnp.maximum(acc_ref[...], contrib)
            acc_ref[...] = jnp.where(rows == g, upd, acc_ref[...])
            return carry

        lax.fori_loop(g_lo, g_hi + 1, gbody, 0)

        @pl.when(i == NTILES - 1)
        def _():
            z_ref[...] = jnp.dot(acc_ref[...], wp_ref[...],
                                 preferred_element_type=jnp.float32) + bp_ref[...]

    return pl.pallas_call(
        body,
        grid=(NTILES,),
        in_specs=[
            pl.BlockSpec((2, B, dinh), lambda i: (0, i, 0)),
            pl.BlockSpec((B, 1), lambda i: (i, 0)),
            pl.BlockSpec((B, 1), lambda i: (i, 0)),
            pl.BlockSpec((2 * dinh, F), lambda i: (0, 0)),
            pl.BlockSpec((1, F), lambda i: (0, 0)),
            pl.BlockSpec((1, 1), lambda i: (0, 0)),
            pl.BlockSpec((F, PROJ), lambda i: (0, 0)),
            pl.BlockSpec((1, PROJ), lambda i: (0, 0)),
        ],
        out_specs=pl.BlockSpec((G, PROJ), lambda i: (0, 0)),
        out_shape=jax.ShapeDtypeStruct((G, PROJ), jnp.float32),
        scratch_shapes=[pltpu.VMEM((G, F), jnp.float32)],
    )(s2, dinv, batchp, w3, b3, a, wp, bp)


def _split_cols(m, dpad):
    m = jnp.pad(m, ((0, 0), (0, dpad - m.shape[1])))
    return m.reshape(P, 2, dpad // 2).transpose(1, 0, 2)


def _split_w(w, b, din_pad, dout_pad):
    w = jnp.pad(w, ((0, din_pad - w.shape[0]), (0, dout_pad - w.shape[1])))
    b = jnp.pad(b, (0, dout_pad - b.shape[0]))
    douth = dout_pad // 2
    w2 = w.reshape(din_pad, 2, douth).transpose(1, 0, 2)
    b2 = b.reshape(1, 2, douth).transpose(1, 0, 2)
    return w2, b2


def kernel(x, edge_index, batch, W1a, b1a, W2a, b2a, W3a, b3a, a1,
           W1b, b1b, W2b, b2b, W3b, b3b, a2, Wp, bp):
    kc, k, rq, lidx, ridx = _consts()
    ka = kb = kh = k

    d1a = max(32, _ceil_to(len(kc), 32))
    d1b = 128
    d2 = 128
    d3 = 224

    pad16 = jnp.full((16,), DUMMY, jnp.int32)
    edges2 = jnp.stack([
        jnp.concatenate([edge_index[0].astype(jnp.int32), pad16]),
        jnp.concatenate([edge_index[1].astype(jnp.int32), pad16]),
    ]).reshape(2, E // 16 + 1, 16)
    srcq, dstq = _sc_compact(edges2, jnp.asarray(lidx), jnp.asarray(ridx),
                             k, rq)
    src_a = jnp.stack([srcq[0], srcq[0] + P])
    src_b = jnp.stack([srcq[1], srcq[1] + P])
    dst_a, dst_b = dstq[0], dstq[1]

    ones = jnp.ones((CH, 16), jnp.float32)
    zeros = jnp.zeros((RPT, 16), jnp.float32)

    cnt = _sc_hist(dstq, ones, zeros, kh)
    cnt_a, cnt_b = cnt[:P], cnt[P:]

    xp = jnp.pad(x, ((0, P - N), (0, 0)))
    xa2 = _split_cols(jnp.take(xp, kc, axis=1), d1a)
    xb2 = _split_cols(xp, d1b)

    g0a, g0b, dinv_a, dinv_b = _tc_prep(xa2, xb2, cnt_a, cnt_b, None, None)

    batchp = jnp.pad(batch.astype(jnp.int32), (0, P - N),
                     constant_values=G).reshape(P, 1)

    w1a2, b1a2 = _split_w(jnp.take(W1a, kc, axis=0), b1a, d1a, d2)
    w1b2, b1b2 = _split_w(W1b, b1b, d1b, d2)
    w2a2, b2a2 = _split_w(W2a, b2a, d2, d3)
    w2b2, b2b2 = _split_w(W2b, b2b, d2, d3)
    a1r = a1.reshape(1, 1)
    a2r = a2.reshape(1, 1)
    w3a = jnp.pad(W3a, ((0, d3 - W3a.shape[0]), (0, 0)))
    w3b = jnp.pad(W3b, ((0, d3 - W3b.shape[0]), (0, 0)))
    b3ar = b3a.reshape(1, -1)
    b3br = b3b.reshape(1, -1)
    wpr = Wp
    bpr = bp.reshape(1, -1)

    def branch(g0, src2, dstb, dinv, w1, b1, w2, b2, w3, b3, ar):
        s1 = _sc_prop(g0.reshape(2 * P, -1), src2, dstb, g0.shape[2], k)
        g1 = _tc_mid(s1.reshape(2, P, -1), dinv, w1, b1, ar, d2)
        s2 = _sc_prop(g1.reshape(2 * P, -1), src2, dstb, d2 // 2, k)
        g2 = _tc_mid(s2.reshape(2, P, -1), dinv, w2, b2, ar, d3)
        s3 = _sc_prop(g2.reshape(2 * P, -1), src2, dstb, d3 // 2, k)
        return _tc_final(s3.reshape(2, P, -1), dinv, batchp, w3, b3, ar,
                         wpr, bpr)

    z1 = branch(g0a, src_a, dst_a, dinv_a, w1a2, b1a2, w2a2, b2a2,
                w3a, b3ar, a1r)
    z2 = branch(g0b, src_b, dst_b, dinv_b, w1b2, b1b2, w2b2, b2b2,
                w3b, b3br, a2r)
    return (z1, z2)


# speedup vs baseline: 1.3066x; 1.1265x over previous
import functools

import numpy as np
import jax
import jax.numpy as jnp
from jax import lax
from jax.experimental import pallas as pl
from jax.experimental.pallas import tpu as pltpu
from jax.experimental.pallas import tpu_sc as plsc

N = 10000
E = 640000
C = 108
G = 128
PROJ = 256

NTILES = 16
RPT = 632
P = NTILES * RPT
CH = 128
PAD_SPREAD = P - N
ROWS_E = (E + PAD_SPREAD) // 16

_CONST = None


def _tf2x32(k1, k2, c1, c2):
    rot0 = (13, 15, 26, 6)
    rot1 = (17, 29, 16, 24)
    ks = (k1, k2, np.uint32(k1 ^ k2 ^ np.uint32(0x1BD11BDA)))
    x0 = (c1 + ks[0]).astype(np.uint32)
    x1 = (c2 + ks[1]).astype(np.uint32)

    def rounds(x0, x1, rots):
        for r in rots:
            x0 = (x0 + x1).astype(np.uint32)
            x1 = ((x1 << np.uint32(r)) | (x1 >> np.uint32(32 - r))).astype(np.uint32)
            x1 = x0 ^ x1
        return x0, x1

    for i, (rots, ka, kb) in enumerate((
            (rot0, ks[1], ks[2]), (rot1, ks[2], ks[0]), (rot0, ks[0], ks[1]),
            (rot1, ks[1], ks[2]), (rot0, ks[2], ks[0]))):
        x0, x1 = rounds(x0, x1, rots)
        x0 = (x0 + ka).astype(np.uint32)
        x1 = (x1 + kb + np.uint32(i + 1)).astype(np.uint32)
    return x0, x1


def _np_uniform01(key, n):
    b1, b2 = _tf2x32(key[0], key[1], np.zeros(n, np.uint32),
                     np.arange(n, dtype=np.uint32))
    bits = b1 ^ b2
    f = ((bits >> np.uint32(9)) | np.uint32(0x3F800000)).view(np.float32)
    return f - np.float32(1.0)


def _consts():
    global _CONST
    if _CONST is not None:
        return _CONST
    b1, b2 = _tf2x32(np.uint32(0), np.uint32(42), np.zeros(4, np.uint32),
                     np.arange(4, dtype=np.uint32))
    subkeys = [(b1[i], b2[i]) for i in range(4)]
    keep1 = _np_uniform01(subkeys[0], E) < 0.5
    keep2 = _np_uniform01(subkeys[1], E) < 0.5
    mask1 = _np_uniform01(subkeys[2], C) < 0.5
    idx1 = np.where(keep1)[0].astype(np.int64)
    idx2 = np.where(keep2)[0].astype(np.int64)
    kc = np.where(~mask1)[0].astype(np.int32)

    k = max(-(-len(i) // (NTILES * CH)) for i in (idx1, idx2))
    oc = k * CH
    spans = []
    pps = []
    for pos in (idx1, idx2):
        flat = np.empty(NTILES * oc, np.int64)
        flat[:len(pos)] = pos
        npad = NTILES * oc - len(pos)
        flat[len(pos):] = E + np.arange(npad) % PAD_SPREAD
        pp = flat.reshape(NTILES, oc)
        spans.append((pp.max(axis=1) // 16 - pp.min(axis=1) // 16 + 1).max())
        pps.append(pp)
    rmax = _ceil_to(int(max(spans)), CH)
    rq = rmax // CH
    lidx = np.empty((2, NTILES, k, CH), np.int32)
    ridx = np.empty((2, NTILES, rq, CH), np.int32)
    for b, pp in enumerate(pps):
        r0p = np.minimum(pp.min(axis=1) // 16, ROWS_E - rmax)
        lidx[b] = (pp - (r0p * 16)[:, None]).reshape(NTILES, k, CH)
        ridx[b] = np.minimum(r0p[:, None] + np.arange(rmax)[None, :],
                             ROWS_E - 1).reshape(NTILES, rq, CH)
    _CONST = (kc, k, rq, lidx, ridx)
    return _CONST


def _ceil_to(v, m):
    return -(-v // m) * m


def _pad_edges(v, k):
    L = NTILES * k * CH
    v = jnp.concatenate([v, jnp.full((L - v.shape[0],), DUMMY, jnp.int32)])
    return v.reshape(NTILES, k, CH)


_MESH = dict(core_axis_name="c", subcore_axis_name="s")
_SC_PARAMS = pltpu.CompilerParams(use_tc_tiling_on_sc=False)
_SC_PARAMS_NOLAYOUT = pltpu.CompilerParams(use_tc_tiling_on_sc=False,
                                           needs_layout_passes=False)


def _sc_compact(edges2, lidx2, ridx2, k, rq):
    rb = rq * CH

    @functools.partial(
        pl.kernel,
        out_type=[jax.ShapeDtypeStruct((2, NTILES, k, CH), jnp.int32),
                  jax.ShapeDtypeStruct((2, NTILES, k, CH), jnp.int32)],
        mesh=plsc.VectorSubcoreMesh(**_MESH),
        compiler_params=_SC_PARAMS_NOLAYOUT,
        scratch_types=[
            pltpu.VMEM((rq, CH), jnp.int32),
            pltpu.VMEM((k, CH), jnp.int32),
            pltpu.VMEM((rb, 16), jnp.int32),
            pltpu.VMEM((k, CH), jnp.int32),
            pltpu.SemaphoreType.DMA,
        ],
    )
    def comp(e_hbm, li_hbm, ri_hbm, src_out, dst_out, vri, vli, reg, vout, sem):
        c = lax.axis_index("c")
        s = lax.axis_index("s")
        pltpu.sync_copy(ri_hbm.at[c].at[s], vri)
        pltpu.sync_copy(li_hbm.at[c].at[s], vli)
        for p, out in ((0, src_out), (1, dst_out)):
            @pl.loop(0, rq)
            def _(q):
                pltpu.async_copy(e_hbm.at[p].at[vri.at[q]],
                                 reg.at[pl.ds(q * CH, CH)], sem).wait()

            @pl.loop(0, k)
            def _(j):
                @pl.loop(0, CH // 16)
                def _(q2):
                    v = vli.at[j][pl.ds(q2 * 16, 16)]
                    row = lax.shift_right_logical(v, 4)
                    lane = lax.bitwise_and(v, 15)
                    vout.at[j][pl.ds(q2 * 16, 16)] = plsc.load_gather(
                        reg, [row, lane])

            pltpu.sync_copy(vout, out.at[c].at[s])

    return comp(edges2, lidx2, ridx2)


def _sc_hist(dst2, ones, zeros, k):
    @functools.partial(
        pl.kernel,
        out_type=jax.ShapeDtypeStruct((2 * P, 16), jnp.float32),
        mesh=plsc.VectorSubcoreMesh(**_MESH),
        compiler_params=_SC_PARAMS,
        scratch_types=[
            pltpu.VMEM((k, CH), jnp.int32),
            pltpu.VMEM((CH, 16), jnp.float32),
            pltpu.VMEM((RPT, 16), jnp.float32),
            pltpu.VMEM_SHARED((P, 16), jnp.float32),
        ],
    )
    def hist(dst_hbm, ones_hbm, zeros_hbm, out_hbm, idst, vones, vzeros, acc):
        c = lax.axis_index("c")
        s = lax.axis_index("s")
        pltpu.sync_copy(dst_hbm.at[c].at[s], idst)
        pltpu.sync_copy(ones_hbm, vones)
        pltpu.sync_copy(zeros_hbm, vzeros)
        pltpu.sync_copy(vzeros, acc.at[pl.ds(s * RPT, RPT)])
        plsc.subcore_barrier()

        @pl.loop(0, k)
        def _(j):
            pltpu.sync_copy(vones, acc.at[idst.at[j]], add=True)

        plsc.subcore_barrier()
        pltpu.sync_copy(acc.at[pl.ds(s * RPT, RPT)],
                        out_hbm.at[pl.ds(c * P + s * RPT, RPT)])

    return hist(dst2, ones, zeros)


def _sc_prop(g2, src2, dstq, dh, k):
    @functools.partial(
        pl.kernel,
        out_type=jax.ShapeDtypeStruct((2 * P, dh), jnp.float32),
        mesh=plsc.VectorSubcoreMesh(**_MESH),
        compiler_params=_SC_PARAMS,
        scratch_types=[
            pltpu.VMEM((k, CH), jnp.int32),
            pltpu.VMEM((k, CH), jnp.int32),
            pltpu.VMEM((CH, dh), jnp.float32),
            pltpu.VMEM_SHARED((P, dh), jnp.float32),
            pltpu.SemaphoreType.DMA,
        ],
    )
    def prop(g_hbm, src_hbm, dst_hbm, out_hbm, isrc, idst, buf, acc, sem):
        c = lax.axis_index("c")
        s = lax.axis_index("s")
        pltpu.sync_copy(src_hbm.at[c].at[s], isrc)
        pltpu.sync_copy(dst_hbm.at[s], idst)
        pltpu.sync_copy(g_hbm.at[pl.ds(c * P + s * RPT, RPT)],
                        acc.at[pl.ds(s * RPT, RPT)])
        plsc.subcore_barrier()

        @pl.loop(0, k)
        def _(j):
            pltpu.async_copy(g_hbm.at[isrc.at[j]], buf, sem).wait()
            pltpu.sync_copy(buf, acc.at[idst.at[j]], add=True)

        plsc.subcore_barrier()
        pltpu.sync_copy(acc.at[pl.ds(s * RPT, RPT)],
                        out_hbm.at[pl.ds(c * P + s * RPT, RPT)])

    return prop(g2, src2, dstq)


def _tc_prep(xa2, xb2, cnt_a, cnt_b, da_out_w, db_out_w):
    B = RPT
    wa = xa2.shape[2]
    wb = xb2.shape[2]

    def body(xa_ref, xb_ref, ca_ref, cb_ref, ga_ref, gb_ref, da_ref, db_ref):
        da = lax.rsqrt(1.0 + ca_ref[:, 0:1])
        db = lax.rsqrt(1.0 + cb_ref[:, 0:1])
        ga_ref[...] = (da * xa_ref[0])[None]
        gb_ref[...] = (db * xb_ref[0])[None]
        da_ref[...] = da
        db_ref[...] = db

    return pl.pallas_call(
        body,
        grid=(2, NTILES),
        in_specs=[
            pl.BlockSpec((1, B, wa), lambda c, i: (c, i, 0)),
            pl.BlockSpec((1, B, wb), lambda c, i: (c, i, 0)),
            pl.BlockSpec((B, 16), lambda c, i: (i, 0)),
            pl.BlockSpec((B, 16), lambda c, i: (i, 0)),
        ],
        out_specs=[
            pl.BlockSpec((1, B, wa), lambda c, i: (c, i, 0)),
            pl.BlockSpec((1, B, wb), lambda c, i: (c, i, 0)),
            pl.BlockSpec((B, 1), lambda c, i: (i, 0)),
            pl.BlockSpec((B, 1), lambda c, i: (i, 0)),
        ],
        out_shape=[
            jax.ShapeDtypeStruct((2, P, wa), jnp.float32),
            jax.ShapeDtypeStruct((2, P, wb), jnp.float32),
            jax.ShapeDtypeStruct((P, 1), jnp.float32),
            jax.ShapeDtypeStruct((P, 1), jnp.float32),
        ],
    )(xa2, xb2, cnt_a, cnt_b)


def _tc_mid(s2, dinv, w2, b2, a, dout):
    B = RPT
    dinh = s2.shape[2]
    din = 2 * dinh
    douth = dout // 2

    def body(s_ref, d_ref, w_ref, b_ref, a_ref, o_ref):
        d = d_ref[...]
        t = jnp.concatenate([s_ref[0], s_ref[1]], axis=1) * d
        y = jnp.dot(t, w_ref[0], preferred_element_type=jnp.float32) + b_ref[0]
        h = jnp.where(y >= 0, y, a_ref[0, 0] * y)
        o_ref[...] = (d * h)[None]

    return pl.pallas_call(
        body,
        grid=(2, NTILES),
        in_specs=[
            pl.BlockSpec((2, B, dinh), lambda c, i: (0, i, 0)),
            pl.BlockSpec((B, 1), lambda c, i: (i, 0)),
            pl.BlockSpec((1, din, douth), lambda c, i: (c, 0, 0)),
            pl.BlockSpec((1, 1, douth), lambda c, i: (c, 0, 0)),
            pl.BlockSpec((1, 1), lambda c, i: (0, 0)),
        ],
        out_specs=pl.BlockSpec((1, B, douth), lambda c, i: (c, i, 0)),
        out_shape=jax.ShapeDtypeStruct((2, P, douth), jnp.float32),
    )(s2, dinv, w2, b2, a)


def _tc_final(s2, dinv, batchp, w3, b3, a, wp, bp):
    B = RPT
    dinh = s2.shape[2]
    F = w3.shape[1]

    def body(s_ref, d_ref, bt_ref, w_ref, b_ref, a_ref, wp_ref, bp_ref,
             z_ref, acc_ref):
        i = pl.program_id(0)

        @pl.when(i == 0)
        def _():
            acc_ref[...] = jnp.full((G, F), -jnp.inf, jnp.float32)

        t = jnp.concatenate([s_ref[0], s_ref[1]], axis=1) * d_ref[...]
        y = jnp.dot(t, w_ref[...], preferred_element_type=jnp.float32) + b_ref[...]
        h = jnp.where(y >= 0, y, a_ref[0, 0] * y)
        bt = bt_ref[...]
        g_lo = bt[0, 0]
        g_hi = bt[B - 1, 0]
        rows = lax.broadcasted_iota(jnp.int32, (G, 1), 0)

        def gbody(g, carry):
            m = bt == g
            contrib = jnp.max(jnp.where(m, h, -jnp.inf), axis=0, keepdims=True)
            upd = jnp.maximum(acc_ref[...], contrib)
            acc_ref[...] = jnp.where(rows == g, upd, acc_ref[...])
            return carry

        lax.fori_loop(g_lo, g_hi + 1, gbody, 0)

        @pl.when(i == NTILES - 1)
        def _():
            z_ref[...] = jnp.dot(acc_ref[...], wp_ref[...],
                                 preferred_element_type=jnp.float32) + bp_ref[...]

    return pl.pallas_call(
        body,
        grid=(NTILES,),
        in_specs=[
            pl.BlockSpec((2, B, dinh), lambda i: (0, i, 0)),
            pl.BlockSpec((B, 1), lambda i: (i, 0)),
            pl.BlockSpec((B, 1), lambda i: (i, 0)),
            pl.BlockSpec((2 * dinh, F), lambda i: (0, 0)),
            pl.BlockSpec((1, F), lambda i: (0, 0)),
            pl.BlockSpec((1, 1), lambda i: (0, 0)),
            pl.BlockSpec((F, PROJ), lambda i: (0, 0)),
            pl.BlockSpec((1, PROJ), lambda i: (0, 0)),
        ],
        out_specs=pl.BlockSpec((G, PROJ), lambda i: (0, 0)),
        out_shape=jax.ShapeDtypeStruct((G, PROJ), jnp.float32),
        scratch_shapes=[pltpu.VMEM((G, F), jnp.float32)],
    )(s2, dinv, batchp, w3, b3, a, wp, bp)


def _split_cols(m, dpad):
    m = jnp.pad(m, ((0, 0), (0, dpad - m.shape[1])))
    return m.reshape(P, 2, dpad // 2).transpose(1, 0, 2)


def _split_w(w, b, din_pad, dout_pad):
    w = jnp.pad(w, ((0, din_pad - w.shape[0]), (0, dout_pad - w.shape[1])))
    b = jnp.pad(b, (0, dout_pad - b.shape[0]))
    douth = dout_pad // 2
    w2 = w.reshape(din_pad, 2, douth).transpose(1, 0, 2)
    b2 = b.reshape(1, 2, douth).transpose(1, 0, 2)
    return w2, b2


def kernel(x, edge_index, batch, W1a, b1a, W2a, b2a, W3a, b3a, a1,
           W1b, b1b, W2b, b2b, W3b, b3b, a2, Wp, bp):
    kc, k, rq, lidx, ridx = _consts()
    ka = kb = kh = k

    d1a = max(32, _ceil_to(len(kc), 32))
    d1b = 128
    d2 = 128
    d3 = 224

    padv = N + jnp.arange(PAD_SPREAD, dtype=jnp.int32)
    edges2 = jnp.stack([
        jnp.concatenate([edge_index[0].astype(jnp.int32), padv]),
        jnp.concatenate([edge_index[1].astype(jnp.int32), padv]),
    ]).reshape(2, ROWS_E, 16)
    srcq, dstq = _sc_compact(edges2, jnp.asarray(lidx), jnp.asarray(ridx),
                             k, rq)
    src_a = jnp.stack([srcq[0], srcq[0] + P])
    src_b = jnp.stack([srcq[1], srcq[1] + P])
    dst_a, dst_b = dstq[0], dstq[1]

    ones = jnp.ones((CH, 16), jnp.float32)
    zeros = jnp.zeros((RPT, 16), jnp.float32)

    cnt = _sc_hist(dstq, ones, zeros, kh)
    cnt_a, cnt_b = cnt[:P], cnt[P:]

    xp = jnp.pad(x, ((0, P - N), (0, 0)))
    xa2 = _split_cols(jnp.take(xp, kc, axis=1), d1a)
    xb2 = _split_cols(xp, d1b)

    g0a, g0b, dinv_a, dinv_b = _tc_prep(xa2, xb2, cnt_a, cnt_b, None, None)

    batchp = jnp.pad(batch.astype(jnp.int32), (0, P - N),
                     constant_values=G).reshape(P, 1)

    w1a2, b1a2 = _split_w(jnp.take(W1a, kc, axis=0), b1a, d1a, d2)
    w1b2, b1b2 = _split_w(W1b, b1b, d1b, d2)
    w2a2, b2a2 = _split_w(W2a, b2a, d2, d3)
    w2b2, b2b2 = _split_w(W2b, b2b, d2, d3)
    a1r = a1.reshape(1, 1)
    a2r = a2.reshape(1, 1)
    w3a = jnp.pad(W3a, ((0, d3 - W3a.shape[0]), (0, 0)))
    w3b = jnp.pad(W3b, ((0, d3 - W3b.shape[0]), (0, 0)))
    b3ar = b3a.reshape(1, -1)
    b3br = b3b.reshape(1, -1)
    wpr = Wp
    bpr = bp.reshape(1, -1)

    def branch(g0, src2, dstb, dinv, w1, b1, w2, b2, w3, b3, ar):
        s1 = _sc_prop(g0.reshape(2 * P, -1), src2, dstb, g0.shape[2], k)
        g1 = _tc_mid(s1.reshape(2, P, -1), dinv, w1, b1, ar, d2)
        s2 = _sc_prop(g1.reshape(2 * P, -1), src2, dstb, d2 // 2, k)
        g2 = _tc_mid(s2.reshape(2, P, -1), dinv, w2, b2, ar, d3)
        s3 = _sc_prop(g2.reshape(2 * P, -1), src2, dstb, d3 // 2, k)
        return _tc_final(s3.reshape(2, P, -1), dinv, batchp, w3, b3, ar,
                         wpr, bpr)

    z1 = branch(g0a, src_a, dst_a, dinv_a, w1a2, b1a2, w2a2, b2a2,
                w3a, b3ar, a1r)
    z2 = branch(g0b, src_b, dst_b, dinv_b, w1b2, b1b2, w2b2, b2b2,
                w3b, b3br, a2r)
    return (z1, z2)


# speedup vs baseline: 1.5340x; 1.1740x over previous
import functools

import numpy as np
import jax
import jax.numpy as jnp
from jax import lax
from jax.experimental import pallas as pl
from jax.experimental.pallas import tpu as pltpu
from jax.experimental.pallas import tpu_sc as plsc

N = 10000
E = 640000
C = 108
G = 128
PROJ = 256

NTILES = 16
RPT = 632
P = NTILES * RPT
CH = 128
RING = 32
PAD_SPREAD = P - N
ROWS_E = (E + PAD_SPREAD) // 16

_CONST = None


def _tf2x32(k1, k2, c1, c2):
    rot0 = (13, 15, 26, 6)
    rot1 = (17, 29, 16, 24)
    ks = (k1, k2, np.uint32(k1 ^ k2 ^ np.uint32(0x1BD11BDA)))
    x0 = (c1 + ks[0]).astype(np.uint32)
    x1 = (c2 + ks[1]).astype(np.uint32)

    def rounds(x0, x1, rots):
        for r in rots:
            x0 = (x0 + x1).astype(np.uint32)
            x1 = ((x1 << np.uint32(r)) | (x1 >> np.uint32(32 - r))).astype(np.uint32)
            x1 = x0 ^ x1
        return x0, x1

    for i, (rots, ka, kb) in enumerate((
            (rot0, ks[1], ks[2]), (rot1, ks[2], ks[0]), (rot0, ks[0], ks[1]),
            (rot1, ks[1], ks[2]), (rot0, ks[2], ks[0]))):
        x0, x1 = rounds(x0, x1, rots)
        x0 = (x0 + ka).astype(np.uint32)
        x1 = (x1 + kb + np.uint32(i + 1)).astype(np.uint32)
    return x0, x1


def _np_uniform01(key, n):
    b1, b2 = _tf2x32(key[0], key[1], np.zeros(n, np.uint32),
                     np.arange(n, dtype=np.uint32))
    bits = b1 ^ b2
    f = ((bits >> np.uint32(9)) | np.uint32(0x3F800000)).view(np.float32)
    return f - np.float32(1.0)


def _consts():
    global _CONST
    if _CONST is not None:
        return _CONST
    b1, b2 = _tf2x32(np.uint32(0), np.uint32(42), np.zeros(4, np.uint32),
                     np.arange(4, dtype=np.uint32))
    subkeys = [(b1[i], b2[i]) for i in range(4)]
    keep1 = _np_uniform01(subkeys[0], E) < 0.5
    keep2 = _np_uniform01(subkeys[1], E) < 0.5
    mask1 = _np_uniform01(subkeys[2], C) < 0.5
    idx1 = np.where(keep1)[0].astype(np.int64)
    idx2 = np.where(keep2)[0].astype(np.int64)
    kc = np.where(~mask1)[0].astype(np.int32)

    k = _ceil_to(max(-(-len(i) // (NTILES * CH)) for i in (idx1, idx2)), RING)
    oc = k * CH
    spans = []
    pps = []
    for pos in (idx1, idx2):
        flat = np.empty(NTILES * oc, np.int64)
        flat[:len(pos)] = pos
        npad = NTILES * oc - len(pos)
        flat[len(pos):] = E + np.arange(npad) % PAD_SPREAD
        pp = flat.reshape(NTILES, oc)
        spans.append((pp.max(axis=1) // 16 - pp.min(axis=1) // 16 + 1).max())
        pps.append(pp)
    rmax = _ceil_to(int(max(spans)), CH)
    rq = rmax // CH
    lidx = np.empty((2, NTILES, k, CH), np.int32)
    ridx = np.empty((2, NTILES, rq, CH), np.int32)
    for b, pp in enumerate(pps):
        r0p = np.minimum(pp.min(axis=1) // 16, ROWS_E - rmax)
        lidx[b] = (pp - (r0p * 16)[:, None]).reshape(NTILES, k, CH)
        ridx[b] = np.minimum(r0p[:, None] + np.arange(rmax)[None, :],
                             ROWS_E - 1).reshape(NTILES, rq, CH)
    _CONST = (kc, k, rq, lidx, ridx)
    return _CONST


def _ceil_to(v, m):
    return -(-v // m) * m


def _pad_edges(v, k):
    L = NTILES * k * CH
    v = jnp.concatenate([v, jnp.full((L - v.shape[0],), DUMMY, jnp.int32)])
    return v.reshape(NTILES, k, CH)


_MESH = dict(core_axis_name="c", subcore_axis_name="s")
_SC_PARAMS = pltpu.CompilerParams(use_tc_tiling_on_sc=False)
_SC_PARAMS_NOLAYOUT = pltpu.CompilerParams(use_tc_tiling_on_sc=False,
                                           needs_layout_passes=False)


def _sc_compact(edges2, lidx2, ridx2, k, rq):
    rb = rq * CH

    @functools.partial(
        pl.kernel,
        out_type=[jax.ShapeDtypeStruct((2, NTILES, k, CH), jnp.int32),
                  jax.ShapeDtypeStruct((2, NTILES, k, CH), jnp.int32)],
        mesh=plsc.VectorSubcoreMesh(**_MESH),
        compiler_params=_SC_PARAMS_NOLAYOUT,
        scratch_types=[
            pltpu.VMEM((rq, CH), jnp.int32),
            pltpu.VMEM((k, CH), jnp.int32),
            pltpu.VMEM((rb, 16), jnp.int32),
            pltpu.VMEM((k, CH), jnp.int32),
            pltpu.SemaphoreType.DMA,
        ],
    )
    def comp(e_hbm, li_hbm, ri_hbm, src_out, dst_out, vri, vli, reg, vout, sem):
        c = lax.axis_index("c")
        s = lax.axis_index("s")
        pltpu.sync_copy(ri_hbm.at[c].at[s], vri)
        pltpu.sync_copy(li_hbm.at[c].at[s], vli)
        for p, out in ((0, src_out), (1, dst_out)):
            @pl.loop(0, rq)
            def _(q):
                pltpu.async_copy(e_hbm.at[p].at[vri.at[q]],
                                 reg.at[pl.ds(q * CH, CH)], sem).wait()

            @pl.loop(0, k)
            def _(j):
                @pl.loop(0, CH // 16)
                def _(q2):
                    v = vli.at[j][pl.ds(q2 * 16, 16)]
                    row = lax.shift_right_logical(v, 4)
                    lane = lax.bitwise_and(v, 15)
                    vout.at[j][pl.ds(q2 * 16, 16)] = plsc.load_gather(
                        reg, [row, lane])

            pltpu.sync_copy(vout, out.at[c].at[s])

    return comp(edges2, lidx2, ridx2)


def _sc_hist(dst2, ones, zeros, k):
    @functools.partial(
        pl.kernel,
        out_type=jax.ShapeDtypeStruct((2 * P, 16), jnp.float32),
        mesh=plsc.VectorSubcoreMesh(**_MESH),
        compiler_params=_SC_PARAMS,
        scratch_types=[
            pltpu.VMEM((k, CH), jnp.int32),
            pltpu.VMEM((CH, 16), jnp.float32),
            pltpu.VMEM((RPT, 16), jnp.float32),
            pltpu.VMEM_SHARED((P, 16), jnp.float32),
        ],
    )
    def hist(dst_hbm, ones_hbm, zeros_hbm, out_hbm, idst, vones, vzeros, acc):
        c = lax.axis_index("c")
        s = lax.axis_index("s")
        pltpu.sync_copy(dst_hbm.at[c].at[s], idst)
        pltpu.sync_copy(ones_hbm, vones)
        pltpu.sync_copy(zeros_hbm, vzeros)
        pltpu.sync_copy(vzeros, acc.at[pl.ds(s * RPT, RPT)])
        plsc.subcore_barrier()

        @pl.loop(0, k)
        def _(j):
            pltpu.sync_copy(vones, acc.at[idst.at[j]], add=True)

        plsc.subcore_barrier()
        pltpu.sync_copy(acc.at[pl.ds(s * RPT, RPT)],
                        out_hbm.at[pl.ds(c * P + s * RPT, RPT)])

    return hist(dst2, ones, zeros)


def _sc_prop(g2, src2, dstq, dh, k):
    @functools.partial(
        pl.kernel,
        out_type=jax.ShapeDtypeStruct((2 * P, dh), jnp.float32),
        mesh=plsc.VectorSubcoreMesh(**_MESH),
        compiler_params=_SC_PARAMS,
        scratch_types=[
            pltpu.VMEM((RING, CH), jnp.int32),
            pltpu.VMEM((RING, CH), jnp.int32),
            pltpu.VMEM((2 * CH, dh), jnp.float32),
            pltpu.VMEM_SHARED((P, dh), jnp.float32),
            pltpu.SemaphoreType.DMA,
            pltpu.SemaphoreType.DMA,
        ],
    )
    def prop(g_hbm, src_hbm, dst_hbm, out_hbm, isrc, idst, buf, acc,
             sg0, sg1):
        buf0 = buf.at[pl.ds(0, CH)]
        buf1 = buf.at[pl.ds(CH, CH)]
        c = lax.axis_index("c")
        s = lax.axis_index("s")
        pltpu.sync_copy(g_hbm.at[pl.ds(c * P + s * RPT, RPT)],
                        acc.at[pl.ds(s * RPT, RPT)])
        plsc.subcore_barrier()

        @pl.loop(0, k // RING)
        def _(gi):
            base = gi * RING
            pltpu.sync_copy(src_hbm.at[c].at[s].at[pl.ds(base, RING)], isrc)
            pltpu.sync_copy(dst_hbm.at[s].at[pl.ds(base, RING)], idst)

            @pl.loop(0, RING, step=2)
            def _(ji):
                cg0 = pltpu.async_copy(g_hbm.at[isrc.at[ji]], buf0, sg0)
                cg1 = pltpu.async_copy(g_hbm.at[isrc.at[ji + 1]], buf1, sg1)
                cg0.wait()
                pltpu.sync_copy(buf0, acc.at[idst.at[ji]], add=True)
                cg1.wait()
                pltpu.sync_copy(buf1, acc.at[idst.at[ji + 1]], add=True)

        plsc.subcore_barrier()
        pltpu.sync_copy(acc.at[pl.ds(s * RPT, RPT)],
                        out_hbm.at[pl.ds(c * P + s * RPT, RPT)])

    return prop(g2, src2, dstq)


def _tc_prep(xa2, xb2, cnt_a, cnt_b, da_out_w, db_out_w):
    B = RPT
    wa = xa2.shape[2]
    wb = xb2.shape[2]

    def body(xa_ref, xb_ref, ca_ref, cb_ref, ga_ref, gb_ref, da_ref, db_ref):
        da = lax.rsqrt(1.0 + ca_ref[:, 0:1])
        db = lax.rsqrt(1.0 + cb_ref[:, 0:1])
        ga_ref[...] = (da * xa_ref[0])[None]
        gb_ref[...] = (db * xb_ref[0])[None]
        da_ref[...] = da
        db_ref[...] = db

    return pl.pallas_call(
        body,
        grid=(2, NTILES),
        in_specs=[
            pl.BlockSpec((1, B, wa), lambda c, i: (c, i, 0)),
            pl.BlockSpec((1, B, wb), lambda c, i: (c, i, 0)),
            pl.BlockSpec((B, 16), lambda c, i: (i, 0)),
            pl.BlockSpec((B, 16), lambda c, i: (i, 0)),
        ],
        out_specs=[
            pl.BlockSpec((1, B, wa), lambda c, i: (c, i, 0)),
            pl.BlockSpec((1, B, wb), lambda c, i: (c, i, 0)),
            pl.BlockSpec((B, 1), lambda c, i: (i, 0)),
            pl.BlockSpec((B, 1), lambda c, i: (i, 0)),
        ],
        out_shape=[
            jax.ShapeDtypeStruct((2, P, wa), jnp.float32),
            jax.ShapeDtypeStruct((2, P, wb), jnp.float32),
            jax.ShapeDtypeStruct((P, 1), jnp.float32),
            jax.ShapeDtypeStruct((P, 1), jnp.float32),
        ],
    )(xa2, xb2, cnt_a, cnt_b)


def _tc_mid(s2, dinv, w2, b2, a, dout):
    B = RPT
    dinh = s2.shape[2]
    din = 2 * dinh
    douth = dout // 2

    def body(s_ref, d_ref, w_ref, b_ref, a_ref, o_ref):
        d = d_ref[...]
        t = jnp.concatenate([s_ref[0], s_ref[1]], axis=1) * d
        y = jnp.dot(t, w_ref[0], preferred_element_type=jnp.float32) + b_ref[0]
        h = jnp.where(y >= 0, y, a_ref[0, 0] * y)
        o_ref[...] = (d * h)[None]

    return pl.pallas_call(
        body,
        grid=(2, NTILES),
        in_specs=[
            pl.BlockSpec((2, B, dinh), lambda c, i: (0, i, 0)),
            pl.BlockSpec((B, 1), lambda c, i: (i, 0)),
            pl.BlockSpec((1, din, douth), lambda c, i: (c, 0, 0)),
            pl.BlockSpec((1, 1, douth), lambda c, i: (c, 0, 0)),
            pl.BlockSpec((1, 1), lambda c, i: (0, 0)),
        ],
        out_specs=pl.BlockSpec((1, B, douth), lambda c, i: (c, i, 0)),
        out_shape=jax.ShapeDtypeStruct((2, P, douth), jnp.float32),
    )(s2, dinv, w2, b2, a)


def _tc_final(s2, dinv, batchp, w3, b3, a, wp, bp):
    B = RPT
    dinh = s2.shape[2]
    F = w3.shape[1]

    def body(s_ref, d_ref, bt_ref, w_ref, b_ref, a_ref, wp_ref, bp_ref,
             z_ref, acc_ref):
        i = pl.program_id(0)

        @pl.when(i == 0)
        def _():
            acc_ref[...] = jnp.full((G, F), -jnp.inf, jnp.float32)

        t = jnp.concatenate([s_ref[0], s_ref[1]], axis=1) * d_ref[...]
        y = jnp.dot(t, w_ref[...], preferred_element_type=jnp.float32) + b_ref[...]
        h = jnp.where(y >= 0, y, a_ref[0, 0] * y)
        bt = bt_ref[...]
        g_lo = bt[0, 0]
        g_hi = bt[B - 1, 0]
        rows = lax.broadcasted_iota(jnp.int32, (G, 1), 0)

        def gbody(g, carry):
            m = bt == g
            contrib = jnp.max(jnp.where(m, h, -jnp.inf), axis=0, keepdims=True)
            upd = jnp.maximum(acc_ref[...], contrib)
            acc_ref[...] = jnp.where(rows == g, upd, acc_ref[...])
            return carry

        lax.fori_loop(g_lo, g_hi + 1, gbody, 0)

        @pl.when(i == NTILES - 1)
        def _():
            z_ref[...] = jnp.dot(acc_ref[...], wp_ref[...],
                                 preferred_element_type=jnp.float32) + bp_ref[...]

    return pl.pallas_call(
        body,
        grid=(NTILES,),
        in_specs=[
            pl.BlockSpec((2, B, dinh), lambda i: (0, i, 0)),
            pl.BlockSpec((B, 1), lambda i: (i, 0)),
            pl.BlockSpec((B, 1), lambda i: (i, 0)),
            pl.BlockSpec((2 * dinh, F), lambda i: (0, 0)),
            pl.BlockSpec((1, F), lambda i: (0, 0)),
            pl.BlockSpec((1, 1), lambda i: (0, 0)),
            pl.BlockSpec((F, PROJ), lambda i: (0, 0)),
            pl.BlockSpec((1, PROJ), lambda i: (0, 0)),
        ],
        out_specs=pl.BlockSpec((G, PROJ), lambda i: (0, 0)),
        out_shape=jax.ShapeDtypeStruct((G, PROJ), jnp.float32),
        scratch_shapes=[pltpu.VMEM((G, F), jnp.float32)],
    )(s2, dinv, batchp, w3, b3, a, wp, bp)


def _split_cols(m, dpad):
    m = jnp.pad(m, ((0, 0), (0, dpad - m.shape[1])))
    return m.reshape(P, 2, dpad // 2).transpose(1, 0, 2)


def _split_w(w, b, din_pad, dout_pad):
    w = jnp.pad(w, ((0, din_pad - w.shape[0]), (0, dout_pad - w.shape[1])))
    b = jnp.pad(b, (0, dout_pad - b.shape[0]))
    douth = dout_pad // 2
    w2 = w.reshape(din_pad, 2, douth).transpose(1, 0, 2)
    b2 = b.reshape(1, 2, douth).transpose(1, 0, 2)
    return w2, b2


def kernel(x, edge_index, batch, W1a, b1a, W2a, b2a, W3a, b3a, a1,
           W1b, b1b, W2b, b2b, W3b, b3b, a2, Wp, bp):
    kc, k, rq, lidx, ridx = _consts()
    ka = kb = kh = k

    d1a = max(32, _ceil_to(len(kc), 32))
    d1b = 128
    d2 = 128
    d3 = 224

    padv = N + jnp.arange(PAD_SPREAD, dtype=jnp.int32)
    edges2 = jnp.stack([
        jnp.concatenate([edge_index[0].astype(jnp.int32), padv]),
        jnp.concatenate([edge_index[1].astype(jnp.int32), padv]),
    ]).reshape(2, ROWS_E, 16)
    srcq, dstq = _sc_compact(edges2, jnp.asarray(lidx), jnp.asarray(ridx),
                             k, rq)
    src_a = jnp.stack([srcq[0], srcq[0] + P])
    src_b = jnp.stack([srcq[1], srcq[1] + P])
    dst_a, dst_b = dstq[0], dstq[1]

    ones = jnp.ones((CH, 16), jnp.float32)
    zeros = jnp.zeros((RPT, 16), jnp.float32)

    cnt = _sc_hist(dstq, ones, zeros, kh)
    cnt_a, cnt_b = cnt[:P], cnt[P:]

    xp = jnp.pad(x, ((0, P - N), (0, 0)))
    xa2 = _split_cols(jnp.take(xp, kc, axis=1), d1a)
    xb2 = _split_cols(xp, d1b)

    g0a, g0b, dinv_a, dinv_b = _tc_prep(xa2, xb2, cnt_a, cnt_b, None, None)

    batchp = jnp.pad(batch.astype(jnp.int32), (0, P - N),
                     constant_values=G).reshape(P, 1)

    w1a2, b1a2 = _split_w(jnp.take(W1a, kc, axis=0), b1a, d1a, d2)
    w1b2, b1b2 = _split_w(W1b, b1b, d1b, d2)
    w2a2, b2a2 = _split_w(W2a, b2a, d2, d3)
    w2b2, b2b2 = _split_w(W2b, b2b, d2, d3)
    a1r = a1.reshape(1, 1)
    a2r = a2.reshape(1, 1)
    w3a = jnp.pad(W3a, ((0, d3 - W3a.shape[0]), (0, 0)))
    w3b = jnp.pad(W3b, ((0, d3 - W3b.shape[0]), (0, 0)))
    b3ar = b3a.reshape(1, -1)
    b3br = b3b.reshape(1, -1)
    wpr = Wp
    bpr = bp.reshape(1, -1)

    def branch(g0, src2, dstb, dinv, w1, b1, w2, b2, w3, b3, ar):
        s1 = _sc_prop(g0.reshape(2 * P, -1), src2, dstb, g0.shape[2], k)
        g1 = _tc_mid(s1.reshape(2, P, -1), dinv, w1, b1, ar, d2)
        s2 = _sc_prop(g1.reshape(2 * P, -1), src2, dstb, d2 // 2, k)
        g2 = _tc_mid(s2.reshape(2, P, -1), dinv, w2, b2, ar, d3)
        s3 = _sc_prop(g2.reshape(2 * P, -1), src2, dstb, d3 // 2, k)
        return _tc_final(s3.reshape(2, P, -1), dinv, batchp, w3, b3, ar,
                         wpr, bpr)

    z1 = branch(g0a, src_a, dst_a, dinv_a, w1a2, b1a2, w2a2, b2a2,
                w3a, b3ar, a1r)
    z2 = branch(g0b, src_b, dst_b, dinv_b, w1b2, b1b2, w2b2, b2b2,
                w3b, b3br, a2r)
    return (z1, z2)


# speedup vs baseline: 1.5347x; 1.0005x over previous
import functools

import numpy as np
import jax
import jax.numpy as jnp
from jax import lax
from jax.experimental import pallas as pl
from jax.experimental.pallas import tpu as pltpu
from jax.experimental.pallas import tpu_sc as plsc

N = 10000
E = 640000
C = 108
G = 128
PROJ = 256

NTILES = 16
RPT = 632
P = NTILES * RPT
CH = 128
RING = 32
PAD_SPREAD = P - N
ROWS_E = (E + PAD_SPREAD) // 16

_CONST = None


def _tf2x32(k1, k2, c1, c2):
    rot0 = (13, 15, 26, 6)
    rot1 = (17, 29, 16, 24)
    ks = (k1, k2, np.uint32(k1 ^ k2 ^ np.uint32(0x1BD11BDA)))
    x0 = (c1 + ks[0]).astype(np.uint32)
    x1 = (c2 + ks[1]).astype(np.uint32)

    def rounds(x0, x1, rots):
        for r in rots:
            x0 = (x0 + x1).astype(np.uint32)
            x1 = ((x1 << np.uint32(r)) | (x1 >> np.uint32(32 - r))).astype(np.uint32)
            x1 = x0 ^ x1
        return x0, x1

    for i, (rots, ka, kb) in enumerate((
            (rot0, ks[1], ks[2]), (rot1, ks[2], ks[0]), (rot0, ks[0], ks[1]),
            (rot1, ks[1], ks[2]), (rot0, ks[2], ks[0]))):
        x0, x1 = rounds(x0, x1, rots)
        x0 = (x0 + ka).astype(np.uint32)
        x1 = (x1 + kb + np.uint32(i + 1)).astype(np.uint32)
    return x0, x1


def _np_uniform01(key, n):
    b1, b2 = _tf2x32(key[0], key[1], np.zeros(n, np.uint32),
                     np.arange(n, dtype=np.uint32))
    bits = b1 ^ b2
    f = ((bits >> np.uint32(9)) | np.uint32(0x3F800000)).view(np.float32)
    return f - np.float32(1.0)


def _consts():
    global _CONST
    if _CONST is not None:
        return _CONST
    b1, b2 = _tf2x32(np.uint32(0), np.uint32(42), np.zeros(4, np.uint32),
                     np.arange(4, dtype=np.uint32))
    subkeys = [(b1[i], b2[i]) for i in range(4)]
    keep1 = _np_uniform01(subkeys[0], E) < 0.5
    keep2 = _np_uniform01(subkeys[1], E) < 0.5
    mask1 = _np_uniform01(subkeys[2], C) < 0.5
    idx1 = np.where(keep1)[0].astype(np.int64)
    idx2 = np.where(keep2)[0].astype(np.int64)
    kc = np.where(~mask1)[0].astype(np.int32)

    k = _ceil_to(max(-(-len(i) // (NTILES * CH)) for i in (idx1, idx2)), RING)
    oc = k * CH
    spans = []
    pps = []
    for pos in (idx1, idx2):
        flat = np.empty(NTILES * oc, np.int64)
        flat[:len(pos)] = pos
        npad = NTILES * oc - len(pos)
        flat[len(pos):] = E + np.arange(npad) % PAD_SPREAD
        pp = flat.reshape(NTILES, oc)
        spans.append((pp.max(axis=1) // 16 - pp.min(axis=1) // 16 + 1).max())
        pps.append(pp)
    rmax = _ceil_to(int(max(spans)), CH)
    rq = rmax // CH
    lidx = np.empty((2, NTILES, k, CH), np.int32)
    ridx = np.empty((2, NTILES, rq, CH), np.int32)
    for b, pp in enumerate(pps):
        r0p = np.minimum(pp.min(axis=1) // 16, ROWS_E - rmax)
        lidx[b] = (pp - (r0p * 16)[:, None]).reshape(NTILES, k, CH)
        ridx[b] = np.minimum(r0p[:, None] + np.arange(rmax)[None, :],
                             ROWS_E - 1).reshape(NTILES, rq, CH)
    _CONST = (kc, k, rq, lidx, ridx)
    return _CONST


def _ceil_to(v, m):
    return -(-v // m) * m


_MESH = dict(core_axis_name="c", subcore_axis_name="s")
_SC_PARAMS = pltpu.CompilerParams(use_tc_tiling_on_sc=False)
_SC_PARAMS_NOLAYOUT = pltpu.CompilerParams(use_tc_tiling_on_sc=False,
                                           needs_layout_passes=False)


def _sc_compact(edges2, lidx2, ridx2, k, rq):
    rb = rq * CH

    @functools.partial(
        pl.kernel,
        out_type=[jax.ShapeDtypeStruct((2, NTILES, k, CH), jnp.int32),
                  jax.ShapeDtypeStruct((2, NTILES, k, CH), jnp.int32)],
        mesh=plsc.VectorSubcoreMesh(**_MESH),
        compiler_params=_SC_PARAMS_NOLAYOUT,
        scratch_types=[
            pltpu.VMEM((rq, CH), jnp.int32),
            pltpu.VMEM((k, CH), jnp.int32),
            pltpu.VMEM((rb, 16), jnp.int32),
            pltpu.VMEM((k, CH), jnp.int32),
            pltpu.SemaphoreType.DMA,
        ],
    )
    def comp(e_hbm, li_hbm, ri_hbm, src_out, dst_out, vri, vli, reg, vout, sem):
        c = lax.axis_index("c")
        s = lax.axis_index("s")
        pltpu.sync_copy(ri_hbm.at[c].at[s], vri)
        pltpu.sync_copy(li_hbm.at[c].at[s], vli)
        for p, out in ((0, src_out), (1, dst_out)):
            @pl.loop(0, rq)
            def _(q):
                pltpu.async_copy(e_hbm.at[p].at[vri.at[q]],
                                 reg.at[pl.ds(q * CH, CH)], sem).wait()

            @pl.loop(0, k)
            def _(j):
                @pl.loop(0, CH // 16)
                def _(q2):
                    v = vli.at[j][pl.ds(q2 * 16, 16)]
                    row = lax.shift_right_logical(v, 4)
                    lane = lax.bitwise_and(v, 15)
                    vout.at[j][pl.ds(q2 * 16, 16)] = plsc.load_gather(
                        reg, [row, lane])

            pltpu.sync_copy(vout, out.at[c].at[s])

    return comp(edges2, lidx2, ridx2)


def _sc_hist(dst2, ones, zeros, k):
    @functools.partial(
        pl.kernel,
        out_type=jax.ShapeDtypeStruct((2 * P, 16), jnp.float32),
        mesh=plsc.VectorSubcoreMesh(**_MESH),
        compiler_params=_SC_PARAMS,
        scratch_types=[
            pltpu.VMEM((k, CH), jnp.int32),
            pltpu.VMEM((CH, 16), jnp.float32),
            pltpu.VMEM((RPT, 16), jnp.float32),
            pltpu.VMEM_SHARED((P, 16), jnp.float32),
        ],
    )
    def hist(dst_hbm, ones_hbm, zeros_hbm, out_hbm, idst, vones, vzeros, acc):
        c = lax.axis_index("c")
        s = lax.axis_index("s")
        pltpu.sync_copy(dst_hbm.at[c].at[s], idst)
        pltpu.sync_copy(ones_hbm, vones)
        pltpu.sync_copy(zeros_hbm, vzeros)
        pltpu.sync_copy(vzeros, acc.at[pl.ds(s * RPT, RPT)])
        plsc.subcore_barrier()

        @pl.loop(0, k)
        def _(j):
            pltpu.sync_copy(vones, acc.at[idst.at[j]], add=True)

        plsc.subcore_barrier()
        pltpu.sync_copy(acc.at[pl.ds(s * RPT, RPT)],
                        out_hbm.at[pl.ds(c * P + s * RPT, RPT)])

    return hist(dst2, ones, zeros)


def _sc_prop(g2, src2, dstq, dh, k):
    @functools.partial(
        pl.kernel,
        out_type=jax.ShapeDtypeStruct((2 * P, dh), jnp.float32),
        mesh=plsc.VectorSubcoreMesh(**_MESH),
        compiler_params=_SC_PARAMS,
        scratch_types=[
            pltpu.VMEM((RING, CH), jnp.int32),
            pltpu.VMEM((RING, CH), jnp.int32),
            pltpu.VMEM((2 * CH, dh), jnp.float32),
            pltpu.VMEM_SHARED((P, dh), jnp.float32),
            pltpu.SemaphoreType.DMA,
            pltpu.SemaphoreType.DMA,
        ],
    )
    def prop(g_hbm, src_hbm, dst_hbm, out_hbm, isrc, idst, buf, acc,
             sg0, sg1):
        buf0 = buf.at[pl.ds(0, CH)]
        buf1 = buf.at[pl.ds(CH, CH)]
        c = lax.axis_index("c")
        s = lax.axis_index("s")
        pltpu.sync_copy(g_hbm.at[pl.ds(c * P + s * RPT, RPT)],
                        acc.at[pl.ds(s * RPT, RPT)])
        plsc.subcore_barrier()

        @pl.loop(0, k // RING)
        def _(gi):
            base = gi * RING
            pltpu.sync_copy(src_hbm.at[c].at[s].at[pl.ds(base, RING)], isrc)
            pltpu.sync_copy(dst_hbm.at[s].at[pl.ds(base, RING)], idst)

            @pl.loop(0, RING, step=2)
            def _(ji):
                cg0 = pltpu.async_copy(g_hbm.at[isrc.at[ji]], buf0, sg0)
                cg1 = pltpu.async_copy(g_hbm.at[isrc.at[ji + 1]], buf1, sg1)
                cg0.wait()
                pltpu.sync_copy(buf0, acc.at[idst.at[ji]], add=True)
                cg1.wait()
                pltpu.sync_copy(buf1, acc.at[idst.at[ji + 1]], add=True)

        plsc.subcore_barrier()
        pltpu.sync_copy(acc.at[pl.ds(s * RPT, RPT)],
                        out_hbm.at[pl.ds(c * P + s * RPT, RPT)])

    return prop(g2, src2, dstq)


def _tc_prep(xa2, xb2, cnt_a, cnt_b):
    B = RPT
    wa = xa2.shape[2]
    wb = xb2.shape[2]

    def body(xa_ref, xb_ref, ca_ref, cb_ref, ga_ref, gb_ref, da_ref, db_ref):
        da = lax.rsqrt(1.0 + ca_ref[:, 0:1])
        db = lax.rsqrt(1.0 + cb_ref[:, 0:1])
        ga_ref[...] = (da * xa_ref[0])[None]
        gb_ref[...] = (db * xb_ref[0])[None]
        da_ref[...] = da
        db_ref[...] = db

    return pl.pallas_call(
        body,
        grid=(2, NTILES),
        in_specs=[
            pl.BlockSpec((1, B, wa), lambda c, i: (c, i, 0)),
            pl.BlockSpec((1, B, wb), lambda c, i: (c, i, 0)),
            pl.BlockSpec((B, 16), lambda c, i: (i, 0)),
            pl.BlockSpec((B, 16), lambda c, i: (i, 0)),
        ],
        out_specs=[
            pl.BlockSpec((1, B, wa), lambda c, i: (c, i, 0)),
            pl.BlockSpec((1, B, wb), lambda c, i: (c, i, 0)),
            pl.BlockSpec((B, 1), lambda c, i: (i, 0)),
            pl.BlockSpec((B, 1), lambda c, i: (i, 0)),
        ],
        out_shape=[
            jax.ShapeDtypeStruct((2, P, wa), jnp.float32),
            jax.ShapeDtypeStruct((2, P, wb), jnp.float32),
            jax.ShapeDtypeStruct((P, 1), jnp.float32),
            jax.ShapeDtypeStruct((P, 1), jnp.float32),
        ],
    )(xa2, xb2, cnt_a, cnt_b)


def _tc_mid(s2, dinv, w2, b2, a, dout):
    B = RPT
    dinh = s2.shape[2]
    din = 2 * dinh
    douth = dout // 2

    def body(s_ref, d_ref, w_ref, b_ref, a_ref, o_ref):
        d = d_ref[...]
        t = jnp.concatenate([s_ref[0], s_ref[1]], axis=1) * d
        y = jnp.dot(t, w_ref[0], preferred_element_type=jnp.float32) + b_ref[0]
        h = jnp.where(y >= 0, y, a_ref[0, 0] * y)
        o_ref[...] = (d * h)[None]

    return pl.pallas_call(
        body,
        grid=(2, NTILES),
        in_specs=[
            pl.BlockSpec((2, B, dinh), lambda c, i: (0, i, 0)),
            pl.BlockSpec((B, 1), lambda c, i: (i, 0)),
            pl.BlockSpec((1, din, douth), lambda c, i: (c, 0, 0)),
            pl.BlockSpec((1, 1, douth), lambda c, i: (c, 0, 0)),
            pl.BlockSpec((1, 1), lambda c, i: (0, 0)),
        ],
        out_specs=pl.BlockSpec((1, B, douth), lambda c, i: (c, i, 0)),
        out_shape=jax.ShapeDtypeStruct((2, P, douth), jnp.float32),
    )(s2, dinv, w2, b2, a)


def _tc_final(s2, dinv, batchp, w3, b3, a, wp, bp):
    B = RPT
    dinh = s2.shape[2]
    F = w3.shape[1]

    def body(s_ref, d_ref, bt_ref, w_ref, b_ref, a_ref, wp_ref, bp_ref,
             z_ref, acc_ref):
        i = pl.program_id(0)

        @pl.when(i == 0)
        def _():
            acc_ref[...] = jnp.full((G, F), -jnp.inf, jnp.float32)

        t = jnp.concatenate([s_ref[0], s_ref[1]], axis=1) * d_ref[...]
        y = jnp.dot(t, w_ref[...], preferred_element_type=jnp.float32) + b_ref[...]
        h = jnp.where(y >= 0, y, a_ref[0, 0] * y)
        bt = bt_ref[...]
        g_lo = bt[0, 0]
        g_hi = bt[B - 1, 0]
        rows = lax.broadcasted_iota(jnp.int32, (G, 1), 0)

        def gbody(g, carry):
            m = bt == g
            contrib = jnp.max(jnp.where(m, h, -jnp.inf), axis=0, keepdims=True)
            upd = jnp.maximum(acc_ref[...], contrib)
            acc_ref[...] = jnp.where(rows == g, upd, acc_ref[...])
            return carry

        lax.fori_loop(g_lo, g_hi + 1, gbody, 0)

        @pl.when(i == NTILES - 1)
        def _():
            z_ref[...] = jnp.dot(acc_ref[...], wp_ref[...],
                                 preferred_element_type=jnp.float32) + bp_ref[...]

    return pl.pallas_call(
        body,
        grid=(NTILES,),
        in_specs=[
            pl.BlockSpec((2, B, dinh), lambda i: (0, i, 0)),
            pl.BlockSpec((B, 1), lambda i: (i, 0)),
            pl.BlockSpec((B, 1), lambda i: (i, 0)),
            pl.BlockSpec((2 * dinh, F), lambda i: (0, 0)),
            pl.BlockSpec((1, F), lambda i: (0, 0)),
            pl.BlockSpec((1, 1), lambda i: (0, 0)),
            pl.BlockSpec((F, PROJ), lambda i: (0, 0)),
            pl.BlockSpec((1, PROJ), lambda i: (0, 0)),
        ],
        out_specs=pl.BlockSpec((G, PROJ), lambda i: (0, 0)),
        out_shape=jax.ShapeDtypeStruct((G, PROJ), jnp.float32),
        scratch_shapes=[pltpu.VMEM((G, F), jnp.float32)],
    )(s2, dinv, batchp, w3, b3, a, wp, bp)


def _split_cols(m, dpad):
    m = jnp.pad(m, ((0, 0), (0, dpad - m.shape[1])))
    return m.reshape(P, 2, dpad // 2).transpose(1, 0, 2)


def _split_w(w, b, din_pad, dout_pad):
    w = jnp.pad(w, ((0, din_pad - w.shape[0]), (0, dout_pad - w.shape[1])))
    b = jnp.pad(b, (0, dout_pad - b.shape[0]))
    douth = dout_pad // 2
    w2 = w.reshape(din_pad, 2, douth).transpose(1, 0, 2)
    b2 = b.reshape(1, 2, douth).transpose(1, 0, 2)
    return w2, b2


def kernel(x, edge_index, batch, W1a, b1a, W2a, b2a, W3a, b3a, a1,
           W1b, b1b, W2b, b2b, W3b, b3b, a2, Wp, bp):
    kc, k, rq, lidx, ridx = _consts()
    ka = kb = kh = k

    d1a = max(32, _ceil_to(len(kc), 32))
    d1b = 128
    d2 = 128
    d3 = 224

    padv = N + jnp.arange(PAD_SPREAD, dtype=jnp.int32)
    edges2 = jnp.stack([
        jnp.concatenate([edge_index[0].astype(jnp.int32), padv]),
        jnp.concatenate([edge_index[1].astype(jnp.int32), padv]),
    ]).reshape(2, ROWS_E, 16)
    srcq, dstq = _sc_compact(edges2, jnp.asarray(lidx), jnp.asarray(ridx),
                             k, rq)
    src_a = jnp.stack([srcq[0], srcq[0] + P])
    src_b = jnp.stack([srcq[1], srcq[1] + P])
    dst_a, dst_b = dstq[0], dstq[1]

    ones = jnp.ones((CH, 16), jnp.float32)
    zeros = jnp.zeros((RPT, 16), jnp.float32)

    cnt = _sc_hist(dstq, ones, zeros, kh)
    cnt_a, cnt_b = cnt[:P], cnt[P:]

    xp = jnp.pad(x, ((0, P - N), (0, 0)))
    xa2 = _split_cols(jnp.take(xp, kc, axis=1), d1a)
    xb2 = _split_cols(xp, d1b)

    g0a, g0b, dinv_a, dinv_b = _tc_prep(xa2, xb2, cnt_a, cnt_b)

    batchp = jnp.pad(batch.astype(jnp.int32), (0, P - N),
                     constant_values=G).reshape(P, 1)

    w1a2, b1a2 = _split_w(jnp.take(W1a, kc, axis=0), b1a, d1a, d2)
    w1b2, b1b2 = _split_w(W1b, b1b, d1b, d2)
    w2a2, b2a2 = _split_w(W2a, b2a, d2, d3)
    w2b2, b2b2 = _split_w(W2b, b2b, d2, d3)
    a1r = a1.reshape(1, 1)
    a2r = a2.reshape(1, 1)
    w3a = jnp.pad(W3a, ((0, d3 - W3a.shape[0]), (0, 0)))
    w3b = jnp.pad(W3b, ((0, d3 - W3b.shape[0]), (0, 0)))
    b3ar = b3a.reshape(1, -1)
    b3br = b3b.reshape(1, -1)
    wpr = Wp
    bpr = bp.reshape(1, -1)

    def branch(g0, src2, dstb, dinv, w1, b1, w2, b2, w3, b3, ar):
        s1 = _sc_prop(g0.reshape(2 * P, -1), src2, dstb, g0.shape[2], k)
        g1 = _tc_mid(s1.reshape(2, P, -1), dinv, w1, b1, ar, d2)
        s2 = _sc_prop(g1.reshape(2 * P, -1), src2, dstb, d2 // 2, k)
        g2 = _tc_mid(s2.reshape(2, P, -1), dinv, w2, b2, ar, d3)
        s3 = _sc_prop(g2.reshape(2 * P, -1), src2, dstb, d3 // 2, k)
        return _tc_final(s3.reshape(2, P, -1), dinv, batchp, w3, b3, ar,
                         wpr, bpr)

    z1 = branch(g0a, src_a, dst_a, dinv_a, w1a2, b1a2, w2a2, b2a2,
                w3a, b3ar, a1r)
    z2 = branch(g0b, src_b, dst_b, dinv_b, w1b2, b1b2, w2b2, b2b2,
                w3b, b3br, a2r)
    return (z1, z2)
